# baseline (speedup 1.0000x reference)
import functools

import jax
import jax.numpy as jnp
from jax import lax
from jax.experimental import pallas as pl
from jax.experimental.pallas import tpu as pltpu
from jax.experimental.pallas import tpu_sc as plsc

_BS, _H, _W, _N, _C, _K = 2, 60, 80, 4096, 64, 3
_M = _H * _W
_MT = 256
_MPAD = 4864
_NW = 32
_RPW = 1024
_RTOT = _NW * _RPW


def _topk_body(p_ref, pb_ref, g_ref, w1_ref, b1_ref, idx_ref, h_ref):
    b = pl.program_id(0)
    t = pl.program_id(1)

    m = lax.broadcasted_iota(jnp.int32, (1, _MT), 1) + t * _MT
    qx = (m % _W).astype(jnp.float32)
    qy = (m // _W).astype(jnp.float32)
    qq = qx * qx + qy * qy

    pxb = pb_ref[0, :, 0:1].astype(jnp.float32)
    pyb = pb_ref[0, :, 1:2].astype(jnp.float32)
    pp = p_ref[0, :, 0:1]

    qp = qx * pxb + qy * pyb
    d2 = (qq - 2.0 * qp) + pp

    fiota = lax.broadcasted_iota(jnp.int32, (_N, _MT), 0).astype(jnp.float32)
    gmat = g_ref[0]

    for k in range(_K):
        minv = jnp.min(d2, axis=0, keepdims=True)
        eq = d2 == minv
        fidx = jnp.min(jnp.where(eq, fiota, jnp.float32(_N)),
                       axis=0, keepdims=True)
        sel = fiota == fidx
        if k + 1 < _K:
            d2 = jnp.where(sel, jnp.float32(3.0e38), d2)

        idx_ref[0, k:k + 1, :] = fidx.astype(jnp.int32) + b * _N

        onehot = jnp.where(sel, jnp.float32(1), jnp.float32(0)).astype(jnp.bfloat16)
        g = lax.dot_general(gmat, onehot, (((1,), (0,)), ((), ())),
                            preferred_element_type=jnp.float32)
        ox = (g[0:1] + g[1:2]) - qx
        oy = (g[2:3] + g[3:4]) - qy
        norm = jnp.sqrt(ox * ox + oy * oy)

        h1 = (w1_ref[:, 0:1] * ox + w1_ref[:, 1:2] * oy
              + w1_ref[:, 2:3] * norm + b1_ref[...])
        h_ref[0, k] = jnp.where(h1 >= 0, h1, 0.1 * h1)


def _sc_gather(tbl_ref, idx_ref, out_ref, idx_v, rows_v, sem):
    wid = lax.axis_index("s") * 2 + lax.axis_index("c")
    pltpu.sync_copy(idx_ref.at[wid], idx_v)
    for h in range(2):
        copies = [pltpu.async_copy(tbl_ref.at[idx_v.at[h * 4 + j]],
                                   rows_v.at[pl.ds(j * 128, 128)], sem)
                  for j in range(4)]
        for c in copies:
            c.wait()
        pltpu.sync_copy(rows_v,
                        out_ref.at[pl.ds(wid * _RPW + h * 512, 512)])


def _combine_body(h_ref, g_ref, w2_ref, b2_ref, w3_ref, b3_ref, out_ref):
    acc = jnp.zeros((_MT, 2 * _C), dtype=jnp.float32)
    for k in range(_K):
        h1 = h_ref[0, k]
        st = lax.dot_general(h1.astype(jnp.bfloat16),
                             w2_ref[...].astype(jnp.bfloat16),
                             (((0,), (1,)), ((), ())),
                             preferred_element_type=jnp.float32)
        s = jax.nn.sigmoid(st + b2_ref[...])
        acc = acc + s * g_ref[0, k]
    out = lax.dot_general(w3_ref[...].astype(jnp.bfloat16),
                          acc.astype(jnp.bfloat16),
                          (((1,), (1,)), ((), ())),
                          preferred_element_type=jnp.float32)
    out = out + b3_ref[...]
    out_ref[0] = jnp.where(out >= 0, out, 0.1 * out)


@jax.jit
def kernel(uv, feat_2d, feat_3d, W1, b1, W2, b2, W3, b3):
    bs = uv.shape[0]
    del feat_2d

    p = jnp.swapaxes(uv, 1, 2)
    pp = jnp.sum(p * p, axis=-1)[..., None]
    pb = p.astype(jnp.bfloat16)

    uv_hi = uv.astype(jnp.bfloat16)
    uv_lo = (uv - uv_hi.astype(jnp.float32)).astype(jnp.bfloat16)
    gmat = jnp.concatenate(
        [uv_hi[:, 0:1], uv_lo[:, 0:1], uv_hi[:, 1:2], uv_lo[:, 1:2],
         jnp.zeros((bs, 4, _N), jnp.bfloat16)], axis=1)

    idx, h4 = pl.pallas_call(
        _topk_body,
        grid=(bs, _MPAD // _MT),
        in_specs=[
            pl.BlockSpec((1, _N, 1), lambda b, t: (b, 0, 0)),
            pl.BlockSpec((1, _N, 2), lambda b, t: (b, 0, 0)),
            pl.BlockSpec((1, 8, _N), lambda b, t: (b, 0, 0)),
            pl.BlockSpec((16, 3), lambda b, t: (0, 0)),
            pl.BlockSpec((16, 1), lambda b, t: (0, 0)),
        ],
        out_specs=[
            pl.BlockSpec((1, _K, _MT), lambda b, t: (b, 0, t)),
            pl.BlockSpec((1, _K, 16, _MT), lambda b, t: (b, 0, 0, t)),
        ],
        out_shape=[
            jax.ShapeDtypeStruct((bs, _K, _MPAD), jnp.int32),
            jax.ShapeDtypeStruct((bs, _K, 16, _MPAD), jnp.float32),
        ],
    )(pp, pb, gmat, W1, b1.reshape(-1, 1))

    tblf = jnp.swapaxes(feat_3d, 1, 2)
    tbl = jnp.concatenate(
        [tblf, jnp.zeros((bs, _N, _C), jnp.float32)], axis=-1
    ).reshape(bs * _N, 2 * _C)
    nflat = bs * _K * _MPAD
    idx3 = jnp.concatenate(
        [idx.reshape(nflat), jnp.zeros((_RTOT - nflat,), jnp.int32)]
    ).reshape(_NW, _RPW // 128, 128)

    mesh = plsc.VectorSubcoreMesh(core_axis_name="c", subcore_axis_name="s")
    rows = pl.kernel(
        _sc_gather,
        out_type=jax.ShapeDtypeStruct((_RTOT, 2 * _C), jnp.float32),
        mesh=mesh,
        scratch_types=[
            pltpu.VMEM((_RPW // 128, 128), jnp.int32),
            pltpu.VMEM((_RPW // 2, 2 * _C), jnp.float32),
            pltpu.SemaphoreType.DMA,
        ],
    )(tbl, idx3)
    g4 = rows[:nflat].reshape(bs, _K, _MPAD, 2 * _C)

    out = pl.pallas_call(
        _combine_body,
        grid=(bs, _MPAD // _MT),
        in_specs=[
            pl.BlockSpec((1, _K, 16, _MT), lambda b, t: (b, 0, 0, t)),
            pl.BlockSpec((1, _K, _MT, 2 * _C), lambda b, t: (b, 0, t, 0)),
            pl.BlockSpec((2 * _C, 16), lambda b, t: (0, 0)),
            pl.BlockSpec((1, 2 * _C), lambda b, t: (0, 0)),
            pl.BlockSpec((_C, 2 * _C), lambda b, t: (0, 0)),
            pl.BlockSpec((_C, 1), lambda b, t: (0, 0)),
        ],
        out_specs=pl.BlockSpec((1, _C, _MT), lambda b, t: (b, 0, t)),
        out_shape=jax.ShapeDtypeStruct((bs, _C, _MPAD), jnp.float32),
    )(h4, g4,
      jnp.concatenate([W2, jnp.zeros((_C, 16), W2.dtype)], axis=0),
      jnp.concatenate([b2, jnp.zeros((_C,), b2.dtype)]).reshape(1, -1),
      jnp.concatenate([W3, jnp.zeros((_C, _C), W3.dtype)], axis=1),
      b3.reshape(-1, 1))

    return out[:, :, :_M].reshape(bs, _C, _H, _W)


# speedup vs baseline: 1.2162x; 1.2162x over previous
import functools

import jax
import jax.numpy as jnp
from jax import lax
from jax.experimental import pallas as pl
from jax.experimental.pallas import tpu as pltpu
from jax.experimental.pallas import tpu_sc as plsc

_BS, _H, _W, _N, _C, _K = 2, 60, 80, 4096, 64, 3
_M = _H * _W
_MT = 256
_MPAD = 4864
_NW = 32
_RPW = 1024
_RTOT = _NW * _RPW


def _topk_body(p_ref, pb_ref, g_ref, w1_ref, b1_ref, idx_ref, h_ref):
    b = pl.program_id(0)
    t = pl.program_id(1)

    m = lax.broadcasted_iota(jnp.int32, (1, _MT), 1) + t * _MT
    qx = (m % _W).astype(jnp.float32)
    qy = (m // _W).astype(jnp.float32)
    qq = qx * qx + qy * qy

    pp = p_ref[0, :, 0:1]

    qmat = jnp.concatenate([qx + qx, qy + qy], axis=0).astype(jnp.bfloat16)
    qp2 = lax.dot_general(pb_ref[0], qmat, (((1,), (0,)), ((), ())),
                          preferred_element_type=jnp.float32)
    d2 = (qq - qp2) + pp

    fiota = lax.broadcasted_iota(jnp.int32, (_N, _MT), 0).astype(jnp.float32)
    gmat = g_ref[0]

    for k in range(_K):
        fidx = jnp.argmin(d2, axis=0).astype(jnp.float32)[None, :]
        sel = fiota == fidx
        if k + 1 < _K:
            d2 = jnp.where(sel, jnp.float32(3.0e38), d2)

        idx_ref[0, k:k + 1, :] = fidx.astype(jnp.int32) + b * _N

        onehot = jnp.where(sel, jnp.float32(1), jnp.float32(0)).astype(jnp.bfloat16)
        g = lax.dot_general(gmat, onehot, (((1,), (0,)), ((), ())),
                            preferred_element_type=jnp.float32)
        ox = (g[0:1] + g[1:2]) - qx
        oy = (g[2:3] + g[3:4]) - qy
        norm = jnp.sqrt(ox * ox + oy * oy)

        h1 = (w1_ref[:, 0:1] * ox + w1_ref[:, 1:2] * oy
              + w1_ref[:, 2:3] * norm + b1_ref[...])
        h_ref[0, k] = jnp.where(h1 >= 0, h1, 0.1 * h1)


def _sc_gather(tbl_ref, idx_ref, out_ref, idx_v, rows_v, sem):
    wid = lax.axis_index("s") * 2 + lax.axis_index("c")
    pltpu.sync_copy(idx_ref.at[wid], idx_v)
    for h in range(2):
        copies = [pltpu.async_copy(tbl_ref.at[idx_v.at[h * 4 + j]],
                                   rows_v.at[pl.ds(j * 128, 128)], sem)
                  for j in range(4)]
        for c in copies:
            c.wait()
        pltpu.sync_copy(rows_v,
                        out_ref.at[pl.ds(wid * _RPW + h * 512, 512)])


def _combine_body(h_ref, g_ref, w2_ref, b2_ref, w3_ref, b3_ref, out_ref):
    acc = jnp.zeros((_MT, 2 * _C), dtype=jnp.float32)
    for k in range(_K):
        h1 = h_ref[0, k]
        st = lax.dot_general(h1.astype(jnp.bfloat16),
                             w2_ref[...].astype(jnp.bfloat16),
                             (((0,), (1,)), ((), ())),
                             preferred_element_type=jnp.float32)
        s = jax.nn.sigmoid(st + b2_ref[...])
        acc = acc + s * g_ref[0, k]
    out = lax.dot_general(w3_ref[...].astype(jnp.bfloat16),
                          acc.astype(jnp.bfloat16),
                          (((1,), (1,)), ((), ())),
                          preferred_element_type=jnp.float32)
    out = out + b3_ref[...]
    out_ref[0] = jnp.where(out >= 0, out, 0.1 * out)


@jax.jit
def kernel(uv, feat_2d, feat_3d, W1, b1, W2, b2, W3, b3):
    bs = uv.shape[0]
    del feat_2d

    p = jnp.swapaxes(uv, 1, 2)
    pp = jnp.sum(p * p, axis=-1)[..., None]
    pb = p.astype(jnp.bfloat16)

    uv_hi = uv.astype(jnp.bfloat16)
    uv_lo = (uv - uv_hi.astype(jnp.float32)).astype(jnp.bfloat16)
    gmat = jnp.concatenate(
        [uv_hi[:, 0:1], uv_lo[:, 0:1], uv_hi[:, 1:2], uv_lo[:, 1:2],
         jnp.zeros((bs, 4, _N), jnp.bfloat16)], axis=1)

    idx, h4 = pl.pallas_call(
        _topk_body,
        grid=(bs, _MPAD // _MT),
        in_specs=[
            pl.BlockSpec((1, _N, 1), lambda b, t: (b, 0, 0)),
            pl.BlockSpec((1, _N, 2), lambda b, t: (b, 0, 0)),
            pl.BlockSpec((1, 8, _N), lambda b, t: (b, 0, 0)),
            pl.BlockSpec((16, 3), lambda b, t: (0, 0)),
            pl.BlockSpec((16, 1), lambda b, t: (0, 0)),
        ],
        out_specs=[
            pl.BlockSpec((1, _K, _MT), lambda b, t: (b, 0, t)),
            pl.BlockSpec((1, _K, 16, _MT), lambda b, t: (b, 0, 0, t)),
        ],
        out_shape=[
            jax.ShapeDtypeStruct((bs, _K, _MPAD), jnp.int32),
            jax.ShapeDtypeStruct((bs, _K, 16, _MPAD), jnp.float32),
        ],
    )(pp, pb, gmat, W1, b1.reshape(-1, 1))

    tblf = jnp.swapaxes(feat_3d, 1, 2)
    tbl = jnp.concatenate(
        [tblf, jnp.zeros((bs, _N, _C), jnp.float32)], axis=-1
    ).reshape(bs * _N, 2 * _C)
    nflat = bs * _K * _MPAD
    idx3 = jnp.concatenate(
        [idx.reshape(nflat), jnp.zeros((_RTOT - nflat,), jnp.int32)]
    ).reshape(_NW, _RPW // 128, 128)

    mesh = plsc.VectorSubcoreMesh(core_axis_name="c", subcore_axis_name="s")
    rows = pl.kernel(
        _sc_gather,
        out_type=jax.ShapeDtypeStruct((_RTOT, 2 * _C), jnp.float32),
        mesh=mesh,
        scratch_types=[
            pltpu.VMEM((_RPW // 128, 128), jnp.int32),
            pltpu.VMEM((_RPW // 2, 2 * _C), jnp.float32),
            pltpu.SemaphoreType.DMA,
        ],
    )(tbl, idx3)
    g4 = rows[:nflat].reshape(bs, _K, _MPAD, 2 * _C)

    out = pl.pallas_call(
        _combine_body,
        grid=(bs, _MPAD // _MT),
        in_specs=[
            pl.BlockSpec((1, _K, 16, _MT), lambda b, t: (b, 0, 0, t)),
            pl.BlockSpec((1, _K, _MT, 2 * _C), lambda b, t: (b, 0, t, 0)),
            pl.BlockSpec((2 * _C, 16), lambda b, t: (0, 0)),
            pl.BlockSpec((1, 2 * _C), lambda b, t: (0, 0)),
            pl.BlockSpec((_C, 2 * _C), lambda b, t: (0, 0)),
            pl.BlockSpec((_C, 1), lambda b, t: (0, 0)),
        ],
        out_specs=pl.BlockSpec((1, _C, _MT), lambda b, t: (b, 0, t)),
        out_shape=jax.ShapeDtypeStruct((bs, _C, _MPAD), jnp.float32),
    )(h4, g4,
      jnp.concatenate([W2, jnp.zeros((_C, 16), W2.dtype)], axis=0),
      jnp.concatenate([b2, jnp.zeros((_C,), b2.dtype)]).reshape(1, -1),
      jnp.concatenate([W3, jnp.zeros((_C, _C), W3.dtype)], axis=1),
      b3.reshape(-1, 1))

    return out[:, :, :_M].reshape(bs, _C, _H, _W)
